# baseline (speedup 1.0000x reference)
import functools

import jax
import jax.numpy as jnp
from jax import lax
from jax.experimental import pallas as pl
from jax.experimental.pallas import tpu as pltpu
from jax.experimental.pallas import tpu_sc as plsc

B = 16
V = 50000
E = 800000
NC = 2
NS = 16
NW = NC * NS
EPT = E // NW
K = 2048
NFULL = EPT // K
TAIL = EPT - NFULL * K
TAILG = (TAIL + 15) // 16
VPT = 1568
VLAST = V - 31 * VPT


def _sc_body(ev_hbm, et_hbm, un_hbm, xt_hbm, out_hbm,
             idx2, idx0b, idx1b, x0r, x1r, tblc, xtv, unic, accb, sem0, sem1):
    wid = lax.axis_index("s") * NC + lax.axis_index("c")
    iota = lax.iota(jnp.int32, 16)

    accb[...] = jnp.zeros((16,), jnp.float32)

    @pl.loop(0, NFULL, init_carry=jnp.zeros((16,), jnp.float32))
    def _chunks(i, acc):
        base = wid * EPT + i * K
        pltpu.sync_copy(ev_hbm.at[pl.ds(2 * base, 2 * K)], idx2)

        @pl.loop(0, K // 16)
        def _deint(g):
            iv = (iota + g * 16) * 2
            idx0b[pl.ds(g * 16, 16)] = plsc.load_gather(idx2, [iv])
            idx1b[pl.ds(g * 16, 16)] = plsc.load_gather(idx2, [iv + 1])

        d0 = pltpu.async_copy(xt_hbm.at[idx0b], x0r, sem0)
        d1 = pltpu.async_copy(xt_hbm.at[idx1b], x1r, sem1)
        pltpu.sync_copy(et_hbm.at[pl.ds(4 * base, 4 * K)], tblc)
        d0.wait()
        d1.wait()

        @pl.loop(0, K, init_carry=jnp.zeros((16,), jnp.float32), unroll=8)
        def _edge(e, a):
            x0v = x0r[e]
            x1v = x1r[e]
            sel = (x0v.astype(jnp.int32) * 2 + x1v.astype(jnp.int32)) + e * 4
            w = plsc.load_gather(tblc, [sel])
            return a + w

        return acc + _edge

    accb[...] = accb[...] + _chunks

    tbase = wid * EPT + NFULL * K
    pltpu.sync_copy(ev_hbm.at[pl.ds(2 * tbase, 2 * TAIL)],
                    idx2.at[pl.ds(0, 2 * TAIL)])

    @pl.loop(0, TAILG)
    def _deint_t(g):
        iv = (iota + g * 16) * 2
        idx0b[pl.ds(g * 16, 16)] = plsc.load_gather(idx2, [iv])
        idx1b[pl.ds(g * 16, 16)] = plsc.load_gather(idx2, [iv + 1])

    d0 = pltpu.async_copy(xt_hbm.at[idx0b], x0r, sem0)
    d1 = pltpu.async_copy(xt_hbm.at[idx1b], x1r, sem1)
    pltpu.sync_copy(et_hbm.at[pl.ds(4 * tbase, 4 * TAIL)],
                    tblc.at[pl.ds(0, 4 * TAIL)])
    d0.wait()
    d1.wait()

    @pl.loop(0, TAIL, init_carry=jnp.zeros((16,), jnp.float32), unroll=8)
    def _edge_t(e, acc):
        x0v = x0r[e]
        x1v = x1r[e]
        sel = (x0v.astype(jnp.int32) * 2 + x1v.astype(jnp.int32)) + e * 4
        w = plsc.load_gather(tblc, [sel])
        return acc + w

    accb[...] = accb[...] + _edge_t

    def uni_phase(vstart, vcnt):
        pltpu.sync_copy(xt_hbm.at[pl.ds(vstart, vcnt)],
                        xtv.at[pl.ds(0, vcnt)])
        pltpu.sync_copy(un_hbm.at[pl.ds(2 * vstart, 2 * vcnt)],
                        unic.at[pl.ds(0, 2 * vcnt)])

        @pl.loop(0, vcnt, init_carry=jnp.zeros((16,), jnp.float32), unroll=8)
        def _uni(v, acc):
            xv = xtv[v]
            uidx = xv.astype(jnp.int32) + 2 * v
            return acc + plsc.load_gather(unic, [uidx])

        accb[...] = accb[...] + _uni

    @pl.when(wid != NW - 1)
    def _():
        uni_phase(wid * VPT, VPT)

    @pl.when(wid == NW - 1)
    def _():
        uni_phase((NW - 1) * VPT, VLAST)

    pltpu.sync_copy(accb, out_hbm.at[wid])


@functools.partial(
    pl.kernel,
    out_type=jax.ShapeDtypeStruct((NW, 16), jnp.float32),
    mesh=plsc.VectorSubcoreMesh(core_axis_name="c", subcore_axis_name="s"),
    compiler_params=pltpu.CompilerParams(
        needs_layout_passes=False, use_tc_tiling_on_sc=False),
    scratch_types=[
        pltpu.VMEM((2 * K,), jnp.int32),
        pltpu.VMEM((K,), jnp.int32),
        pltpu.VMEM((K,), jnp.int32),
        pltpu.VMEM((K, 16), jnp.float32),
        pltpu.VMEM((K, 16), jnp.float32),
        pltpu.VMEM((4 * K,), jnp.float32),
        pltpu.VMEM((VPT, 16), jnp.float32),
        pltpu.VMEM((2 * VPT,), jnp.float32),
        pltpu.VMEM((16,), jnp.float32),
        pltpu.SemaphoreType.DMA,
        pltpu.SemaphoreType.DMA,
    ],
)
def _mn_edges(ev_hbm, et_hbm, un_hbm, xt_hbm, out_hbm, *scratch):
    _sc_body(ev_hbm, et_hbm, un_hbm, xt_hbm, out_hbm, *scratch)


def kernel(x, univariate_vars, univariate_tables, bivariate_vars, bivariate_tables):
    del univariate_vars
    ev = bivariate_vars.astype(jnp.int32).reshape(-1)
    et = bivariate_tables.reshape(-1)
    un = univariate_tables.reshape(-1)
    xt = x.T
    partials = _mn_edges(ev, et, un, xt)
    return jnp.sum(partials, axis=0)


# speedup vs baseline: 10.7607x; 10.7607x over previous
import functools

import jax
import jax.numpy as jnp
from jax import lax
from jax.experimental import pallas as pl
from jax.experimental.pallas import tpu as pltpu
from jax.experimental.pallas import tpu_sc as plsc

B = 16
V = 50000
E = 800000
NC = 2
NS = 16
NW = NC * NS
NB = E // 128
BPT_LO = NB // NW
NHI = NB - BPT_LO * NW
CB = 16
NFULL = BPT_LO // CB
TAIL_LO = BPT_LO - NFULL * CB
TAIL_HI = TAIL_LO + 1
VPT = 1568
VLAST = V - 31 * VPT


def _sc_body(ev_hbm, et_hbm, un_hbm, xt_hbm, out_hbm,
             slab, x0r, x1r, tblc, xtv, unic, accb, sem0, sem1):
    wid = lax.axis_index("s") * NC + lax.axis_index("c")
    blk0 = BPT_LO * wid + jnp.minimum(wid, NHI)

    accb[...] = jnp.zeros((16,), jnp.float32)

    def edge_chunk(bs, nb):
        pltpu.sync_copy(ev_hbm.at[pl.ds(bs, nb)], slab.at[pl.ds(0, nb)])
        ds = []
        for b in range(nb):
            ds.append(pltpu.async_copy(
                xt_hbm.at[slab.at[b, 0]], x0r.at[pl.ds(b * 128, 128)], sem0))
            ds.append(pltpu.async_copy(
                xt_hbm.at[slab.at[b, 1]], x1r.at[pl.ds(b * 128, 128)], sem1))
        pltpu.sync_copy(et_hbm.at[0, pl.ds(bs, nb)], tblc.at[0, pl.ds(0, nb)])
        pltpu.sync_copy(et_hbm.at[1, pl.ds(bs, nb)], tblc.at[1, pl.ds(0, nb)])
        for d in ds:
            d.wait()

        @pl.loop(0, nb * 128, init_carry=jnp.zeros((16,), jnp.float32),
                 unroll=8)
        def _edge(e, acc):
            x0v = x0r[e]
            x1v = x1r[e]
            bi = jnp.broadcast_to(e >> 7, (16,))
            li = jnp.broadcast_to(e & 127, (16,))
            w = plsc.load_gather(
                tblc, [x0v.astype(jnp.int32), bi, x1v.astype(jnp.int32), li])
            return acc + w

        return _edge

    @pl.loop(0, NFULL, init_carry=jnp.zeros((16,), jnp.float32))
    def _chunks(i, acc):
        return acc + edge_chunk(blk0 + i * CB, CB)

    accb[...] = accb[...] + _chunks
    tbs = blk0 + NFULL * CB

    @pl.when(wid < NHI)
    def _():
        accb[...] = accb[...] + edge_chunk(tbs, TAIL_HI)

    @pl.when(wid >= NHI)
    def _():
        accb[...] = accb[...] + edge_chunk(tbs, TAIL_LO)

    def uni_phase(vstart, vcnt):
        pltpu.sync_copy(xt_hbm.at[pl.ds(vstart, vcnt)], xtv.at[pl.ds(0, vcnt)])
        pltpu.sync_copy(un_hbm.at[pl.ds(2 * vstart, 2 * vcnt)],
                        unic.at[pl.ds(0, 2 * vcnt)])

        @pl.loop(0, vcnt, init_carry=jnp.zeros((16,), jnp.float32), unroll=8)
        def _uni(v, acc):
            xv = xtv[v]
            uidx = xv.astype(jnp.int32) + 2 * v
            return acc + plsc.load_gather(unic, [uidx])

        accb[...] = accb[...] + _uni

    @pl.when(wid != NW - 1)
    def _():
        uni_phase(wid * VPT, VPT)

    @pl.when(wid == NW - 1)
    def _():
        uni_phase((NW - 1) * VPT, VLAST)

    pltpu.sync_copy(accb, out_hbm.at[wid])


@functools.partial(
    pl.kernel,
    out_type=jax.ShapeDtypeStruct((NW, 16), jnp.float32),
    mesh=plsc.VectorSubcoreMesh(core_axis_name="c", subcore_axis_name="s"),
    compiler_params=pltpu.CompilerParams(
        needs_layout_passes=False, use_tc_tiling_on_sc=False),
    scratch_types=[
        pltpu.VMEM((CB, 2, 128), jnp.int32),
        pltpu.VMEM((CB * 128, 16), jnp.float32),
        pltpu.VMEM((CB * 128, 16), jnp.float32),
        pltpu.VMEM((2, CB, 2, 128), jnp.float32),
        pltpu.VMEM((VPT, 16), jnp.float32),
        pltpu.VMEM((2 * VPT,), jnp.float32),
        pltpu.VMEM((16,), jnp.float32),
        pltpu.SemaphoreType.DMA,
        pltpu.SemaphoreType.DMA,
    ],
)
def _mn_edges(ev_hbm, et_hbm, un_hbm, xt_hbm, out_hbm, *scratch):
    _sc_body(ev_hbm, et_hbm, un_hbm, xt_hbm, out_hbm, *scratch)


def kernel(x, univariate_vars, univariate_tables, bivariate_vars, bivariate_tables):
    del univariate_vars
    ev = bivariate_vars.reshape(NB, 128, 2).transpose(0, 2, 1)
    et = bivariate_tables.reshape(NB, 128, 2, 2).transpose(2, 0, 3, 1)
    un = univariate_tables.reshape(-1)
    xt = x.T
    partials = _mn_edges(ev, et, un, xt)
    return jnp.sum(partials, axis=0)


# speedup vs baseline: 11.1454x; 1.0358x over previous
import functools

import jax
import jax.numpy as jnp
from jax import lax
from jax.experimental import pallas as pl
from jax.experimental.pallas import tpu as pltpu
from jax.experimental.pallas import tpu_sc as plsc

B = 16
V = 50000
E = 800000
NC = 2
NS = 16
NW = NC * NS
NB = E // 128
BPT_LO = NB // NW
NHI = NB - BPT_LO * NW
CB = 8
NFULL = BPT_LO // CB
TAIL_LO = BPT_LO - NFULL * CB
TAIL_HI = TAIL_LO + 1
KE = CB * 128
VPT = 1568
VLAST = V - 31 * VPT


def _sc_body(ev_hbm, et_hbm, un_hbm, xt_hbm, out_hbm,
             slab, x0r, x1r, tblc, xtv, unic, accb, sem0, sem1):
    wid = lax.axis_index("s") * NC + lax.axis_index("c")
    blk0 = BPT_LO * wid + jnp.minimum(wid, NHI)

    accb[...] = jnp.zeros((16,), jnp.float32)

    def stage(bs, pp, sem):
        pltpu.sync_copy(ev_hbm.at[pl.ds(bs, CB)], slab.at[pp])
        for b in range(CB):
            pltpu.async_copy(xt_hbm.at[slab.at[pp, b, 0]],
                             x0r.at[pp, pl.ds(b * 128, 128)], sem)
            pltpu.async_copy(xt_hbm.at[slab.at[pp, b, 1]],
                             x1r.at[pp, pl.ds(b * 128, 128)], sem)
        pltpu.sync_copy(et_hbm.at[pl.ds(bs * 256, 2 * KE)],
                        tblc.at[pp, pl.ds(0, 2 * KE)])
        pltpu.sync_copy(et_hbm.at[pl.ds(E * 2 + bs * 256, 2 * KE)],
                        tblc.at[pp, pl.ds(2 * KE, 2 * KE)])

    def drain(pp, sem):
        pltpu.make_async_copy(xt_hbm.at[pl.ds(0, KE)], x0r.at[pp], sem).wait()
        pltpu.make_async_copy(xt_hbm.at[pl.ds(0, KE)], x1r.at[pp], sem).wait()

    def compute(pp, nedges):
        @pl.loop(0, nedges, init_carry=jnp.zeros((16,), jnp.float32),
                 unroll=8)
        def _edge(e, acc):
            x0v = x0r[pp, e]
            x1v = x1r[pp, e]
            ofs = e + (e & -128)
            sel = (x0v * float(2 * KE) + x1v * 128.0).astype(jnp.int32) + ofs
            w = plsc.load_gather(tblc.at[pp], [sel])
            return acc + w

        return _edge

    stage(blk0, 0, sem0)

    @pl.loop(0, NFULL, init_carry=jnp.zeros((16,), jnp.float32))
    def _chunks(i, acc):
        p = i & 1

        @pl.when(i < NFULL - 1)
        def _():
            @pl.when(p == 0)
            def _():
                stage(blk0 + (i + 1) * CB, 1, sem1)

            @pl.when(p == 1)
            def _():
                stage(blk0 + (i + 1) * CB, 0, sem0)

        @pl.when(p == 0)
        def _():
            drain(0, sem0)

        @pl.when(p == 1)
        def _():
            drain(1, sem1)

        return acc + compute(p, KE)

    accb[...] = accb[...] + _chunks

    tbs = blk0 + NFULL * CB

    def tail(nb):
        pltpu.sync_copy(ev_hbm.at[pl.ds(tbs, nb)], slab.at[0, pl.ds(0, nb)])
        for b in range(nb):
            pltpu.async_copy(xt_hbm.at[slab.at[0, b, 0]],
                             x0r.at[0, pl.ds(b * 128, 128)], sem0)
            pltpu.async_copy(xt_hbm.at[slab.at[0, b, 1]],
                             x1r.at[0, pl.ds(b * 128, 128)], sem0)
        pltpu.sync_copy(et_hbm.at[pl.ds(tbs * 256, nb * 256)],
                        tblc.at[0, pl.ds(0, nb * 256)])
        pltpu.sync_copy(et_hbm.at[pl.ds(E * 2 + tbs * 256, nb * 256)],
                        tblc.at[0, pl.ds(nb * 256, nb * 256)])
        pltpu.make_async_copy(xt_hbm.at[pl.ds(0, nb * 128)],
                              x0r.at[0, pl.ds(0, nb * 128)], sem0).wait()
        pltpu.make_async_copy(xt_hbm.at[pl.ds(0, nb * 128)],
                              x1r.at[0, pl.ds(0, nb * 128)], sem0).wait()

        @pl.loop(0, nb * 128, init_carry=jnp.zeros((16,), jnp.float32),
                 unroll=8)
        def _edge(e, acc):
            x0v = x0r[0, e]
            x1v = x1r[0, e]
            ofs = e + (e & -128)
            sel = (x0v * float(2 * nb * 128) + x1v * 128.0).astype(jnp.int32) + ofs
            w = plsc.load_gather(tblc.at[0], [sel])
            return acc + w

        accb[...] = accb[...] + _edge

    @pl.when(wid < NHI)
    def _():
        tail(TAIL_HI)

    @pl.when(wid >= NHI)
    def _():
        tail(TAIL_LO)

    def uni_phase(vstart, vcnt):
        pltpu.sync_copy(xt_hbm.at[pl.ds(vstart, vcnt)], xtv.at[pl.ds(0, vcnt)])
        pltpu.sync_copy(un_hbm.at[pl.ds(2 * vstart, 2 * vcnt)],
                        unic.at[pl.ds(0, 2 * vcnt)])

        @pl.loop(0, vcnt, init_carry=jnp.zeros((16,), jnp.float32), unroll=8)
        def _uni(v, acc):
            xv = xtv[v]
            uidx = xv.astype(jnp.int32) + 2 * v
            return acc + plsc.load_gather(unic, [uidx])

        accb[...] = accb[...] + _uni

    @pl.when(wid != NW - 1)
    def _():
        uni_phase(wid * VPT, VPT)

    @pl.when(wid == NW - 1)
    def _():
        uni_phase((NW - 1) * VPT, VLAST)

    pltpu.sync_copy(accb, out_hbm.at[wid])


@functools.partial(
    pl.kernel,
    out_type=jax.ShapeDtypeStruct((NW, 16), jnp.float32),
    mesh=plsc.VectorSubcoreMesh(core_axis_name="c", subcore_axis_name="s"),
    compiler_params=pltpu.CompilerParams(
        needs_layout_passes=False, use_tc_tiling_on_sc=False),
    scratch_types=[
        pltpu.VMEM((2, CB, 2, 128), jnp.int32),
        pltpu.VMEM((2, KE, 16), jnp.float32),
        pltpu.VMEM((2, KE, 16), jnp.float32),
        pltpu.VMEM((2, 4 * KE), jnp.float32),
        pltpu.VMEM((VPT, 16), jnp.float32),
        pltpu.VMEM((2 * VPT,), jnp.float32),
        pltpu.VMEM((16,), jnp.float32),
        pltpu.SemaphoreType.DMA,
        pltpu.SemaphoreType.DMA,
    ],
)
def _mn_edges(ev_hbm, et_hbm, un_hbm, xt_hbm, out_hbm, *scratch):
    _sc_body(ev_hbm, et_hbm, un_hbm, xt_hbm, out_hbm, *scratch)


def kernel(x, univariate_vars, univariate_tables, bivariate_vars, bivariate_tables):
    del univariate_vars
    ev = bivariate_vars.reshape(NB, 128, 2).transpose(0, 2, 1)
    et = bivariate_tables.reshape(NB, 128, 2, 2).transpose(2, 0, 3, 1).reshape(-1)
    un = univariate_tables.reshape(-1)
    xt = x.T
    partials = _mn_edges(ev, et, un, xt)
    return jnp.sum(partials, axis=0)


# speedup vs baseline: 12.6682x; 1.1366x over previous
import functools

import jax
import jax.numpy as jnp
from jax import lax
from jax.experimental import pallas as pl
from jax.experimental.pallas import tpu as pltpu
from jax.experimental.pallas import tpu_sc as plsc

B = 16
V = 50000
E = 800000
NC = 2
NS = 16
NW = NC * NS
NB = E // 128
BPT_LO = NB // NW
NHI = NB - BPT_LO * NW
CB = 8
NFULL = BPT_LO // CB
TAIL_LO = BPT_LO - NFULL * CB
TAIL_HI = TAIL_LO + 1
KE = CB * 128
VPT = 1568
VLAST = V - 31 * VPT


def _sc_body(ev_hbm, et_hbm, un_hbm, xt_hbm, out_hbm,
             slab, x0r, x1r, tblc, xtv, unic, accb, sem0, sem1, semS):
    wid = lax.axis_index("s") * NC + lax.axis_index("c")
    blk0 = BPT_LO * wid + jnp.minimum(wid, NHI)

    accb[...] = jnp.zeros((16,), jnp.float32)

    def enqueue(bs, pp, sem):
        for b in range(CB):
            pltpu.async_copy(xt_hbm.at[slab.at[pp, b, 0]],
                             x0r.at[pp, pl.ds(b * 128, 128)], sem)
            pltpu.async_copy(xt_hbm.at[slab.at[pp, b, 1]],
                             x1r.at[pp, pl.ds(b * 128, 128)], sem)
        pltpu.async_copy(et_hbm.at[pl.ds(bs * 256, 2 * KE)],
                         tblc.at[pp, pl.ds(0, 2 * KE)], sem)
        pltpu.async_copy(et_hbm.at[pl.ds(E * 2 + bs * 256, 2 * KE)],
                         tblc.at[pp, pl.ds(2 * KE, 2 * KE)], sem)

    def drain(pp, sem):
        pltpu.make_async_copy(xt_hbm.at[pl.ds(0, KE)], x0r.at[pp], sem).wait()
        pltpu.make_async_copy(xt_hbm.at[pl.ds(0, KE)], x1r.at[pp], sem).wait()
        pltpu.make_async_copy(et_hbm.at[pl.ds(0, 2 * KE)],
                              tblc.at[pp, pl.ds(0, 2 * KE)], sem).wait()
        pltpu.make_async_copy(et_hbm.at[pl.ds(0, 2 * KE)],
                              tblc.at[pp, pl.ds(2 * KE, 2 * KE)], sem).wait()

    def wait_slab(pp):
        pltpu.make_async_copy(ev_hbm.at[pl.ds(0, CB)], slab.at[pp], semS).wait()

    ZERO4 = (jnp.zeros((16,), jnp.float32),) * 4

    def compute(pp, nedges):
        @pl.loop(0, nedges, init_carry=ZERO4, unroll=8)
        def _edge(e, accs):
            a0, a1, a2, a3 = accs
            x0v = x0r[pp, e]
            x1v = x1r[pp, e]
            ofs = e + (e & -128)
            sel = (x0v * float(2 * KE) + x1v * 128.0).astype(jnp.int32) + ofs
            w = plsc.load_gather(tblc.at[pp], [sel])
            return (a1, a2, a3, a0 + w)

        a0, a1, a2, a3 = _edge
        return (a0 + a1) + (a2 + a3)

    pltpu.sync_copy(ev_hbm.at[pl.ds(blk0, CB)], slab.at[0])
    enqueue(blk0, 0, sem0)
    pltpu.async_copy(ev_hbm.at[pl.ds(blk0 + CB, CB)], slab.at[1], semS)

    @pl.loop(0, NFULL, init_carry=jnp.zeros((16,), jnp.float32))
    def _chunks(i, acc):
        p = i & 1

        @pl.when(i < NFULL - 1)
        def _():
            @pl.when(p == 0)
            def _():
                wait_slab(1)
                enqueue(blk0 + (i + 1) * CB, 1, sem1)

            @pl.when(p == 1)
            def _():
                wait_slab(0)
                enqueue(blk0 + (i + 1) * CB, 0, sem0)

        @pl.when(p == 0)
        def _():
            drain(0, sem0)

        @pl.when(p == 1)
        def _():
            drain(1, sem1)

        @pl.when(i < NFULL - 2)
        def _():
            @pl.when(p == 0)
            def _():
                pltpu.async_copy(ev_hbm.at[pl.ds(blk0 + (i + 2) * CB, CB)],
                                 slab.at[0], semS)

            @pl.when(p == 1)
            def _():
                pltpu.async_copy(ev_hbm.at[pl.ds(blk0 + (i + 2) * CB, CB)],
                                 slab.at[1], semS)

        return acc + compute(p, KE)

    accb[...] = accb[...] + _chunks

    tbs = blk0 + NFULL * CB

    def tail(nb):
        pltpu.sync_copy(ev_hbm.at[pl.ds(tbs, nb)], slab.at[0, pl.ds(0, nb)])
        for b in range(nb):
            pltpu.async_copy(xt_hbm.at[slab.at[0, b, 0]],
                             x0r.at[0, pl.ds(b * 128, 128)], sem0)
            pltpu.async_copy(xt_hbm.at[slab.at[0, b, 1]],
                             x1r.at[0, pl.ds(b * 128, 128)], sem0)
        pltpu.sync_copy(et_hbm.at[pl.ds(tbs * 256, nb * 256)],
                        tblc.at[0, pl.ds(0, nb * 256)])
        pltpu.sync_copy(et_hbm.at[pl.ds(E * 2 + tbs * 256, nb * 256)],
                        tblc.at[0, pl.ds(nb * 256, nb * 256)])
        pltpu.make_async_copy(xt_hbm.at[pl.ds(0, nb * 128)],
                              x0r.at[0, pl.ds(0, nb * 128)], sem0).wait()
        pltpu.make_async_copy(xt_hbm.at[pl.ds(0, nb * 128)],
                              x1r.at[0, pl.ds(0, nb * 128)], sem0).wait()

        @pl.loop(0, nb * 128, init_carry=ZERO4, unroll=8)
        def _edge(e, accs):
            a0, a1, a2, a3 = accs
            x0v = x0r[0, e]
            x1v = x1r[0, e]
            ofs = e + (e & -128)
            sel = (x0v * float(2 * nb * 128) + x1v * 128.0).astype(jnp.int32) + ofs
            w = plsc.load_gather(tblc.at[0], [sel])
            return (a1, a2, a3, a0 + w)

        a0, a1, a2, a3 = _edge
        accb[...] = accb[...] + ((a0 + a1) + (a2 + a3))

    @pl.when(wid < NHI)
    def _():
        tail(TAIL_HI)

    @pl.when(wid >= NHI)
    def _():
        tail(TAIL_LO)

    def uni_phase(vstart, vcnt):
        pltpu.sync_copy(xt_hbm.at[pl.ds(vstart, vcnt)], xtv.at[pl.ds(0, vcnt)])
        pltpu.sync_copy(un_hbm.at[pl.ds(2 * vstart, 2 * vcnt)],
                        unic.at[pl.ds(0, 2 * vcnt)])

        @pl.loop(0, vcnt, init_carry=ZERO4, unroll=8)
        def _uni(v, accs):
            a0, a1, a2, a3 = accs
            xv = xtv[v]
            uidx = xv.astype(jnp.int32) + 2 * v
            return (a1, a2, a3, a0 + plsc.load_gather(unic, [uidx]))

        a0, a1, a2, a3 = _uni
        accb[...] = accb[...] + ((a0 + a1) + (a2 + a3))

    @pl.when(wid != NW - 1)
    def _():
        uni_phase(wid * VPT, VPT)

    @pl.when(wid == NW - 1)
    def _():
        uni_phase((NW - 1) * VPT, VLAST)

    pltpu.sync_copy(accb, out_hbm.at[wid])


@functools.partial(
    pl.kernel,
    out_type=jax.ShapeDtypeStruct((NW, 16), jnp.float32),
    mesh=plsc.VectorSubcoreMesh(core_axis_name="c", subcore_axis_name="s"),
    compiler_params=pltpu.CompilerParams(
        needs_layout_passes=False, use_tc_tiling_on_sc=False),
    scratch_types=[
        pltpu.VMEM((2, CB, 2, 128), jnp.int32),
        pltpu.VMEM((2, KE, 16), jnp.float32),
        pltpu.VMEM((2, KE, 16), jnp.float32),
        pltpu.VMEM((2, 4 * KE), jnp.float32),
        pltpu.VMEM((VPT, 16), jnp.float32),
        pltpu.VMEM((2 * VPT,), jnp.float32),
        pltpu.VMEM((16,), jnp.float32),
        pltpu.SemaphoreType.DMA,
        pltpu.SemaphoreType.DMA,
        pltpu.SemaphoreType.DMA,
    ],
)
def _mn_edges(ev_hbm, et_hbm, un_hbm, xt_hbm, out_hbm, *scratch):
    _sc_body(ev_hbm, et_hbm, un_hbm, xt_hbm, out_hbm, *scratch)


def kernel(x, univariate_vars, univariate_tables, bivariate_vars, bivariate_tables):
    del univariate_vars
    ev = bivariate_vars.reshape(NB, 128, 2).transpose(0, 2, 1)
    et = bivariate_tables.reshape(NB, 128, 2, 2).transpose(2, 0, 3, 1).reshape(-1)
    un = univariate_tables.reshape(-1)
    xt = x.T
    partials = _mn_edges(ev, et, un, xt)
    return jnp.sum(partials, axis=0)


# speedup vs baseline: 22.1501x; 1.7485x over previous
import functools

import jax
import jax.numpy as jnp
from jax import lax
from jax.experimental import pallas as pl
from jax.experimental.pallas import tpu as pltpu
from jax.experimental.pallas import tpu_sc as plsc

B = 16
V = 50000
E = 800000
NC = 2
NS = 16
NW = NC * NS
NB = E // 128
WPB = V // 16
BPT_LO = NB // NW
NHI = NB - BPT_LO * NW
CB = 16
NFULL = BPT_LO // CB
TAIL_LO = BPT_LO - NFULL * CB
TAIL_HI = TAIL_LO + 1
KE = CB * 128
VPT = 1568
VLAST = V - 31 * VPT


def _sc_body(ev_hbm, et_hbm, un_hbm, xw_hbm, out_hbm,
             slab, tblc, xwv, unic, accv, sem0, sem1):
    wid = lax.axis_index("s") * NC + lax.axis_index("c")
    blk0 = BPT_LO * wid + jnp.minimum(wid, NHI)
    iota = lax.iota(jnp.int32, 16)

    pltpu.sync_copy(xw_hbm, xwv)

    def stage(bs, pp, sem):
        pltpu.async_copy(ev_hbm.at[pl.ds(bs, CB)], slab.at[pp], sem)
        pltpu.async_copy(et_hbm.at[pl.ds(bs * 256, 2 * KE)],
                         tblc.at[pp, pl.ds(0, 2 * KE)], sem)
        pltpu.async_copy(et_hbm.at[pl.ds(E * 2 + bs * 256, 2 * KE)],
                         tblc.at[pp, pl.ds(2 * KE, 2 * KE)], sem)

    def drain(pp, sem):
        pltpu.make_async_copy(ev_hbm.at[pl.ds(0, CB)], slab.at[pp], sem).wait()
        pltpu.make_async_copy(et_hbm.at[pl.ds(0, 2 * KE)],
                              tblc.at[pp, pl.ds(0, 2 * KE)], sem).wait()
        pltpu.make_async_copy(et_hbm.at[pl.ds(0, 2 * KE)],
                              tblc.at[pp, pl.ds(2 * KE, 2 * KE)], sem).wait()

    ZERO16 = (jnp.zeros((16,), jnp.float32),) * 16

    def compute(pp, nb, accs0):
        plane = nb * 256

        @pl.loop(0, nb * 8, init_carry=accs0)
        def _grp(g, accs):
            blk = g >> 3
            l0 = (g & 7) * 16
            lbase = iota + (blk * 256 + l0)
            idx0 = slab[pp, blk, 0, pl.ds(l0, 16)]
            idx1 = slab[pp, blk, 1, pl.ds(l0, 16)]
            wa0 = idx0 >> 4
            sa0 = idx0 & 15
            wa1 = idx1 >> 4
            sa1 = idx1 & 15
            out = []
            for b in range(16):
                w0 = plsc.load_gather(xwv.at[b], [wa0])
                w1 = plsc.load_gather(xwv.at[b], [wa1])
                bit0 = (w0 >> sa0) & 1
                bit1 = (w1 >> sa1) & 1
                addr = (bit0 * plane + bit1 * 128) + lbase
                out.append(accs[b] + plsc.load_gather(tblc.at[pp], [addr]))
            return tuple(out)

        return _grp

    stage(blk0, 0, sem0)

    @pl.loop(0, NFULL, init_carry=ZERO16)
    def _chunks(i, accs):
        p = i & 1

        @pl.when(i < NFULL - 1)
        def _():
            @pl.when(p == 0)
            def _():
                stage(blk0 + (i + 1) * CB, 1, sem1)

            @pl.when(p == 1)
            def _():
                stage(blk0 + (i + 1) * CB, 0, sem0)

        @pl.when(p == 0)
        def _():
            drain(0, sem0)

        @pl.when(p == 1)
        def _():
            drain(1, sem1)

        return compute(p, CB, accs)

    for b in range(16):
        accv[b] = _chunks[b]

    tbs = blk0 + NFULL * CB

    def tail(nb):
        pltpu.sync_copy(ev_hbm.at[pl.ds(tbs, nb)], slab.at[0, pl.ds(0, nb)])
        pltpu.sync_copy(et_hbm.at[pl.ds(tbs * 256, nb * 256)],
                        tblc.at[0, pl.ds(0, nb * 256)])
        pltpu.sync_copy(et_hbm.at[pl.ds(E * 2 + tbs * 256, nb * 256)],
                        tblc.at[0, pl.ds(nb * 256, nb * 256)])
        accs = compute(0, nb, tuple(accv[b] for b in range(16)))
        for b in range(16):
            accv[b] = accs[b]

    @pl.when(wid < NHI)
    def _():
        tail(TAIL_HI)

    @pl.when(wid >= NHI)
    def _():
        tail(TAIL_LO)

    iota2 = iota * 2

    def uni_phase(vstart, vcnt):
        pltpu.sync_copy(un_hbm.at[pl.ds(2 * vstart, 2 * vcnt)],
                        unic.at[pl.ds(0, 2 * vcnt)])
        w0 = vstart >> 4

        @pl.loop(0, vcnt // 16,
                 init_carry=tuple(accv[b] for b in range(16)))
        def _uni(j, accs):
            widx = jnp.broadcast_to(w0 + j, (16,))
            out = []
            for b in range(16):
                wv = plsc.load_gather(xwv.at[b], [widx])
                bit = (wv >> iota) & 1
                uidx = (iota2 + bit) + 32 * j
                out.append(accs[b] + plsc.load_gather(unic, [uidx]))
            return tuple(out)

        for b in range(16):
            accv[b] = _uni[b]

    @pl.when(wid != NW - 1)
    def _():
        uni_phase(wid * VPT, VPT)

    @pl.when(wid == NW - 1)
    def _():
        uni_phase((NW - 1) * VPT, VLAST)

    outv = jnp.zeros((16,), jnp.float32)
    for b in range(16):
        tot = jnp.sum(accv[b], axis=0)
        outv = jnp.where(iota == b, tot, outv)
    accv[0] = outv
    pltpu.sync_copy(accv.at[0], out_hbm.at[wid])


@functools.partial(
    pl.kernel,
    out_type=jax.ShapeDtypeStruct((NW, 16), jnp.float32),
    mesh=plsc.VectorSubcoreMesh(core_axis_name="c", subcore_axis_name="s"),
    compiler_params=pltpu.CompilerParams(
        needs_layout_passes=False, use_tc_tiling_on_sc=False),
    scratch_types=[
        pltpu.VMEM((2, CB, 2, 128), jnp.int32),
        pltpu.VMEM((2, 4 * KE), jnp.float32),
        pltpu.VMEM((B, WPB), jnp.int32),
        pltpu.VMEM((2 * VPT,), jnp.float32),
        pltpu.VMEM((16, 16), jnp.float32),
        pltpu.SemaphoreType.DMA,
        pltpu.SemaphoreType.DMA,
    ],
)
def _mn_edges(ev_hbm, et_hbm, un_hbm, xw_hbm, out_hbm, *scratch):
    _sc_body(ev_hbm, et_hbm, un_hbm, xw_hbm, out_hbm, *scratch)


def kernel(x, univariate_vars, univariate_tables, bivariate_vars, bivariate_tables):
    del univariate_vars
    pow2 = 2.0 ** jnp.arange(16, dtype=jnp.float32)
    xw = jnp.dot(x.reshape(B, WPB, 16), pow2).astype(jnp.int32)
    ev = bivariate_vars.reshape(NB, 128, 2).transpose(0, 2, 1)
    et = bivariate_tables.reshape(NB, 128, 2, 2).transpose(2, 0, 3, 1).reshape(-1)
    un = univariate_tables.reshape(-1)
    partials = _mn_edges(ev, et, un, xw)
    return jnp.sum(partials, axis=0)
